# Initial kernel scaffold; baseline (speedup 1.0000x reference)
import jax
import jax.numpy as jnp
from jax.experimental import pallas as pl


def kernel(atom_fea, nbr_fea, self_fea_idx, nbr_fea_idx, crystal_atom_idx, W_emb, b_emb, W_full, b_full, bn1_g, bn1_b, bn2_g, bn2_b):
    raise NotImplementedError("write your pallas kernel here")


# speedup vs baseline: 1.0458x; 1.0458x over previous
import functools

import jax
import jax.numpy as jnp
from jax import lax
from jax.experimental import pallas as pl
from jax.experimental.pallas import tpu as pltpu
from jax.experimental.pallas import tpu_sc as plsc

N = 50000
E = 800000
EMB = 128
FEA = 64
HID = 128
NBRF = 16
NCRYS = 1000
EPS = 1e-5

NC, NS = 2, 16
NW = NC * NS
CH = 128
EPAD = 802816
EPW = EPAD // NW
NCHUNK = EPW // CH
NPAD = 53248
APW = NPAD // NW
ACHUNK = APW // CH
NTAB = 51200
NROWS_PT = NTAB // NS
CTAB = 1024
CROWS_PT = CTAB // NS

_mesh = lambda: plsc.VectorSubcoreMesh(core_axis_name="c", subcore_axis_name="s")


def _wid():
    return lax.axis_index("s") * NC + lax.axis_index("c")


def _zero_vmem(ref, nrow, ncol):
    def row(r, carry):
        for k in range(ncol // 16):
            ref[r, pl.ds(k * 16, 16)] = jnp.zeros((16,), jnp.float32)
        return carry
    lax.fori_loop(0, nrow, row, 0)


@functools.partial(
    pl.kernel,
    mesh=_mesh(),
    out_type=jax.ShapeDtypeStruct((EPAD, HID), jnp.float32),
    scratch_types=[
        pltpu.VMEM((CH,), jnp.int32),
        pltpu.VMEM((CH,), jnp.int32),
        pltpu.VMEM((CH, HID), jnp.float32),
        pltpu.VMEM((CH, HID), jnp.float32),
        pltpu.VMEM((CH, HID), jnp.float32),
        pltpu.SemaphoreType.DMA,
        pltpu.SemaphoreType.DMA,
    ],
)
def _sc_gather(xs_h, xn_h, nprj_h, sidx_h, nidx_h, t_h,
               sidx_v, nidx_v, av, bv, pv, sem1, sem2):
    base = _wid() * EPW

    def chunk(c, carry):
        e0 = base + c * CH
        pltpu.sync_copy(sidx_h.at[pl.ds(e0, CH)], sidx_v)
        pltpu.sync_copy(nidx_h.at[pl.ds(e0, CH)], nidx_v)
        cp1 = pltpu.async_copy(xs_h.at[sidx_v], av, sem1)
        cp2 = pltpu.async_copy(xn_h.at[nidx_v], bv, sem2)
        pltpu.sync_copy(nprj_h.at[pl.ds(e0, CH)], pv)
        cp1.wait()
        cp2.wait()

        def row(r, rc):
            for k in range(HID // 16):
                s = pl.ds(k * 16, 16)
                pv[r, s] = pv[r, s] + av[r, s] + bv[r, s]
            return rc
        lax.fori_loop(0, CH, row, 0)
        pltpu.sync_copy(pv, t_h.at[pl.ds(e0, CH)])
        return carry

    lax.fori_loop(0, NCHUNK, chunk, 0)


@functools.partial(
    pl.kernel,
    mesh=_mesh(),
    out_type=jax.ShapeDtypeStruct((4, NTAB, FEA // 4), jnp.float32),
    scratch_types=[
        pltpu.VMEM((CH,), jnp.int32),
        pltpu.VMEM((CH,), jnp.int32),
        pltpu.VMEM((CH, FEA // 4), jnp.float32),
        pltpu.VMEM((CH, FEA // 4), jnp.float32),
        pltpu.VMEM_SHARED((NTAB, FEA // 4), jnp.float32),
    ],
)
def _sc_scatter(msg_h, sidx_h, ramp_h, out_h, idx2, ridx, mv, zv, tab):
    cid = lax.axis_index("c")
    sid = lax.axis_index("s")
    base = sid * (EPAD // NS)

    _zero_vmem(zv, CH, FEA // 4)
    for q in range(2):
        quarter = cid * 2 + q
        for j in range(NROWS_PT // CH):
            pltpu.sync_copy(ramp_h.at[pl.ds(sid * NROWS_PT + j * CH, CH)], ridx)
            pltpu.sync_copy(zv, tab.at[ridx], add=False)
        plsc.subcore_barrier()

        def chunk(c, carry):
            e0 = base + c * CH
            pltpu.sync_copy(sidx_h.at[pl.ds(e0, CH)], idx2)
            pltpu.sync_copy(msg_h.at[quarter, pl.ds(e0, CH)], mv)
            pltpu.sync_copy(mv, tab.at[idx2], add=True)
            return carry

        lax.fori_loop(0, EPAD // NS // CH, chunk, 0)
        plsc.subcore_barrier()

        for j in range(NROWS_PT // CH):
            r0 = sid * NROWS_PT + j * CH
            pltpu.sync_copy(ramp_h.at[pl.ds(r0, CH)], ridx)
            pltpu.sync_copy(tab.at[ridx], mv)
            pltpu.sync_copy(mv, out_h.at[quarter, pl.ds(r0, CH)])
        plsc.subcore_barrier()


@functools.partial(
    pl.kernel,
    mesh=_mesh(),
    out_type=(jax.ShapeDtypeStruct((NC, NTAB, 16), jnp.float32),
              jax.ShapeDtypeStruct((NC, CTAB, 16), jnp.float32)),
    scratch_types=[
        pltpu.VMEM((CH,), jnp.int32),
        pltpu.VMEM((CH,), jnp.int32),
        pltpu.VMEM((CROWS_PT,), jnp.int32),
        pltpu.VMEM((CH, 16), jnp.float32),
        pltpu.VMEM((CH, 16), jnp.float32),
        pltpu.VMEM((CROWS_PT, 16), jnp.float32),
        pltpu.VMEM_SHARED((NTAB, 16), jnp.float32),
        pltpu.VMEM_SHARED((CTAB, 16), jnp.float32),
    ],
)
def _sc_counts(sidx_h, cidx_h, ramp_h, ecnt_h, ccnt_h,
               idx2, ridx, cridx, ones_v, zv, czv, etab, ctab):
    cid = lax.axis_index("c")
    sid = lax.axis_index("s")

    def orow(r, carry):
        ones_v[r, pl.ds(0, 16)] = jnp.full((16,), 1.0, jnp.float32)
        return carry
    lax.fori_loop(0, CH, orow, 0)
    _zero_vmem(zv, CH, 16)
    _zero_vmem(czv, CROWS_PT, 16)
    for j in range(NROWS_PT // CH):
        pltpu.sync_copy(ramp_h.at[pl.ds(sid * NROWS_PT + j * CH, CH)], ridx)
        pltpu.sync_copy(zv, etab.at[ridx], add=False)
    pltpu.sync_copy(ramp_h.at[pl.ds(sid * CROWS_PT, CROWS_PT)], cridx)
    pltpu.sync_copy(czv, ctab.at[cridx], add=False)
    plsc.subcore_barrier()

    ebase = (cid * NS + sid) * EPW

    def echunk(c, carry):
        pltpu.sync_copy(sidx_h.at[pl.ds(ebase + c * CH, CH)], idx2)
        pltpu.sync_copy(ones_v, etab.at[idx2], add=True)
        return carry
    lax.fori_loop(0, NCHUNK, echunk, 0)

    abase = (cid * NS + sid) * APW

    def achunk(c, carry):
        pltpu.sync_copy(cidx_h.at[pl.ds(abase + c * CH, CH)], idx2)
        pltpu.sync_copy(ones_v, ctab.at[idx2], add=True)
        return carry
    lax.fori_loop(0, ACHUNK, achunk, 0)
    plsc.subcore_barrier()

    for j in range(NROWS_PT // CH):
        r0 = sid * NROWS_PT + j * CH
        pltpu.sync_copy(ramp_h.at[pl.ds(r0, CH)], ridx)
        pltpu.sync_copy(etab.at[ridx], zv)
        pltpu.sync_copy(zv, ecnt_h.at[cid, pl.ds(r0, CH)])
    r0 = sid * CROWS_PT
    pltpu.sync_copy(ramp_h.at[pl.ds(r0, CROWS_PT)], cridx)
    pltpu.sync_copy(ctab.at[cridx], czv)
    pltpu.sync_copy(czv, ccnt_h.at[cid, pl.ds(r0, CROWS_PT)])


@functools.partial(
    pl.kernel,
    mesh=_mesh(),
    out_type=jax.ShapeDtypeStruct((NC, CTAB, FEA // 2), jnp.float32),
    scratch_types=[
        pltpu.VMEM((CH,), jnp.int32),
        pltpu.VMEM((CROWS_PT,), jnp.int32),
        pltpu.VMEM((CH, FEA // 2), jnp.float32),
        pltpu.VMEM((CROWS_PT, FEA // 2), jnp.float32),
        pltpu.VMEM_SHARED((CTAB, FEA // 2), jnp.float32),
    ],
)
def _sc_crys(x_h, cidx_h, ramp_h, out_h, idx2, cridx, mv, czv, tab):
    cid = lax.axis_index("c")
    sid = lax.axis_index("s")

    _zero_vmem(czv, CROWS_PT, FEA // 2)
    r0 = sid * CROWS_PT
    pltpu.sync_copy(ramp_h.at[pl.ds(r0, CROWS_PT)], cridx)
    pltpu.sync_copy(czv, tab.at[cridx], add=False)
    plsc.subcore_barrier()

    base = sid * (NPAD // NS)

    def chunk(c, carry):
        e0 = base + c * CH
        pltpu.sync_copy(cidx_h.at[pl.ds(e0, CH)], idx2)
        pltpu.sync_copy(x_h.at[cid, pl.ds(e0, CH)], mv)
        pltpu.sync_copy(mv, tab.at[idx2], add=True)
        return carry

    lax.fori_loop(0, NPAD // NS // CH, chunk, 0)
    plsc.subcore_barrier()

    pltpu.sync_copy(tab.at[cridx], czv)
    pltpu.sync_copy(czv, out_h.at[cid, pl.ds(r0, CROWS_PT)])


def _embed_body(af, w, b, o):
    o[...] = jnp.dot(af[...], w[...], preferred_element_type=jnp.float32) + b[...]


def _tc_embed(atom_fea, W_emb, b_emb):
    return pl.pallas_call(
        _embed_body,
        grid=(50,),
        in_specs=[pl.BlockSpec((1000, EMB), lambda i: (i, 0)),
                  pl.BlockSpec((EMB, FEA), lambda i: (0, 0)),
                  pl.BlockSpec((1, FEA), lambda i: (0, 0))],
        out_specs=pl.BlockSpec((1000, FEA), lambda i: (i, 0)),
        out_shape=jax.ShapeDtypeStruct((N, FEA), jnp.float32),
    )(atom_fea, W_emb, b_emb.reshape(1, FEA))


def _proj_body(x, ws, wn, os, on):
    xv = x[...]
    os[...] = jnp.dot(xv, ws[...], preferred_element_type=jnp.float32)
    on[...] = jnp.dot(xv, wn[...], preferred_element_type=jnp.float32)


def _tc_proj(x, Ws, Wn):
    return pl.pallas_call(
        _proj_body,
        grid=(50,),
        in_specs=[pl.BlockSpec((1000, FEA), lambda i: (i, 0)),
                  pl.BlockSpec((FEA, HID), lambda i: (0, 0)),
                  pl.BlockSpec((FEA, HID), lambda i: (0, 0))],
        out_specs=[pl.BlockSpec((1000, HID), lambda i: (i, 0)),
                   pl.BlockSpec((1000, HID), lambda i: (i, 0))],
        out_shape=[jax.ShapeDtypeStruct((N, HID), jnp.float32),
                   jax.ShapeDtypeStruct((N, HID), jnp.float32)],
    )(x, Ws, Wn)


def _nprj_body(nf, w, b, o):
    o[...] = jnp.dot(nf[...], w[...], preferred_element_type=jnp.float32) + b[...]


def _tc_nprj(nbr_fea_p, Wp, bf):
    return pl.pallas_call(
        _nprj_body,
        grid=(392,),
        in_specs=[pl.BlockSpec((2048, NBRF), lambda i: (i, 0)),
                  pl.BlockSpec((NBRF, HID), lambda i: (0, 0)),
                  pl.BlockSpec((1, HID), lambda i: (0, 0))],
        out_specs=pl.BlockSpec((2048, HID), lambda i: (i, 0)),
        out_shape=jax.ShapeDtypeStruct((EPAD, HID), jnp.float32),
    )(nbr_fea_p, Wp, bf.reshape(1, HID))


_SBLK = 2000
_SGRID = E // _SBLK


def _stats_body(t, g, b, o, acc):
    i = pl.program_id(0)

    @pl.when(i == 0)
    def _():
        acc[...] = jnp.zeros_like(acc)

    x = t[...]
    acc[0:1, :] += jnp.sum(x, axis=0, keepdims=True)
    acc[1:2, :] += jnp.sum(x * x, axis=0, keepdims=True)

    @pl.when(i == _SGRID - 1)
    def _():
        mu = acc[0:1, :] / E
        var = acc[1:2, :] / E - mu * mu
        a = g[...] * lax.rsqrt(var + EPS)
        o[0:1, :] = a
        o[1:2, :] = b[...] - mu * a


def _tc_stats(t, g1, b1):
    return pl.pallas_call(
        _stats_body,
        grid=(_SGRID,),
        in_specs=[pl.BlockSpec((_SBLK, HID), lambda i: (i, 0)),
                  pl.BlockSpec((1, HID), lambda i: (0, 0)),
                  pl.BlockSpec((1, HID), lambda i: (0, 0))],
        out_specs=pl.BlockSpec((2, HID), lambda i: (0, 0)),
        out_shape=jax.ShapeDtypeStruct((2, HID), jnp.float32),
        scratch_shapes=[pltpu.VMEM((2, HID), jnp.float32)],
    )(t, g1.reshape(1, HID), b1.reshape(1, HID))


_MBLK = 2048
_MGRID = EPAD // _MBLK


def _msg_body(t, ac, o):
    i = pl.program_id(0)
    y = t[...] * ac[0:1, :] + ac[1:2, :]
    m = jax.nn.sigmoid(y[:, :FEA]) * jax.nn.softplus(y[:, FEA:])
    rows = i * _MBLK + lax.broadcasted_iota(jnp.int32, (_MBLK, 1), 0)
    m = jnp.where(rows < E, m, 0.0)
    for q in range(4):
        o[q, :, :] = m[:, q * (FEA // 4):(q + 1) * (FEA // 4)]


def _tc_msg(t, ac):
    return pl.pallas_call(
        _msg_body,
        grid=(_MGRID,),
        in_specs=[pl.BlockSpec((_MBLK, HID), lambda i: (i, 0)),
                  pl.BlockSpec((2, HID), lambda i: (0, 0))],
        out_specs=pl.BlockSpec((4, _MBLK, FEA // 4), lambda i: (0, i, 0)),
        out_shape=jax.ShapeDtypeStruct((4, EPAD, FEA // 4), jnp.float32),
    )(t, ac)


_UBLK = 1000
_UGRID = N // _UBLK


def _upd_body(s, parts, x, g, b, o, acc):
    p = pl.program_id(0)
    i = pl.program_id(1)
    cnt = parts[0, :, 0] + parts[1, :, 0]
    sv = jnp.concatenate([s[q, :, :] for q in range(4)], axis=1)
    mean = sv / jnp.clip(cnt, 1.0, None)[:, None]

    @pl.when((p == 0) & (i == 0))
    def _():
        acc[...] = jnp.zeros_like(acc)

    @pl.when(p == 0)
    def _():
        acc[0:1, :] += jnp.sum(mean, axis=0, keepdims=True)
        acc[1:2, :] += jnp.sum(mean * mean, axis=0, keepdims=True)
        o[...] = mean

    @pl.when((p == 1) & (i == 0))
    def _():
        mu = acc[0:1, :] / N
        var = acc[1:2, :] / N - mu * mu
        a = g[...] * lax.rsqrt(var + EPS)
        acc[0:1, :] = a
        acc[1:2, :] = b[...] - mu * a

    @pl.when(p == 1)
    def _():
        o[...] = jax.nn.softplus(x[...] + mean * acc[0:1, :] + acc[1:2, :])


def _tc_update(sums, cnt_parts, x, g2, b2):
    return pl.pallas_call(
        _upd_body,
        grid=(2, _UGRID),
        in_specs=[pl.BlockSpec((4, _UBLK, FEA // 4), lambda p, i: (0, i, 0)),
                  pl.BlockSpec((NC, _UBLK, 16), lambda p, i: (0, i, 0)),
                  pl.BlockSpec((_UBLK, FEA), lambda p, i: (i, 0)),
                  pl.BlockSpec((1, FEA), lambda p, i: (0, 0)),
                  pl.BlockSpec((1, FEA), lambda p, i: (0, 0))],
        out_specs=pl.BlockSpec((_UBLK, FEA), lambda p, i: (i, 0)),
        out_shape=jax.ShapeDtypeStruct((N, FEA), jnp.float32),
        scratch_shapes=[pltpu.VMEM((2, FEA), jnp.float32)],
    )(sums, cnt_parts, x, g2.reshape(1, FEA), b2.reshape(1, FEA))


def _crys_body(s, parts, o):
    cnt = parts[0, :, 0] + parts[1, :, 0]
    sv = jnp.concatenate([s[0, :, :], s[1, :, :]], axis=1)
    o[...] = jax.nn.softplus(sv / jnp.clip(cnt, 1.0, None)[:, None])


def _tc_crys(csums, ccnt_parts):
    return pl.pallas_call(
        _crys_body,
        grid=(1,),
        in_specs=[pl.BlockSpec((NC, NCRYS, FEA // 2), lambda i: (0, 0, 0)),
                  pl.BlockSpec((NC, NCRYS, 16), lambda i: (0, 0, 0))],
        out_specs=pl.BlockSpec((NCRYS, FEA), lambda i: (0, 0)),
        out_shape=jax.ShapeDtypeStruct((NCRYS, FEA), jnp.float32),
    )(csums, ccnt_parts)


def kernel(atom_fea, nbr_fea, self_fea_idx, nbr_fea_idx, crystal_atom_idx,
           W_emb, b_emb, W_full, b_full, bn1_g, bn1_b, bn2_g, bn2_b):
    sidx_g = jnp.zeros((EPAD,), jnp.int32).at[:E].set(self_fea_idx)
    nidx_g = jnp.zeros((EPAD,), jnp.int32).at[:E].set(nbr_fea_idx)
    nbr_fea_p = jnp.zeros((EPAD, NBRF), jnp.float32).at[:E].set(nbr_fea)

    cnt = jax.ops.segment_sum(jnp.ones((E,), jnp.float32), self_fea_idx,
                              num_segments=N)
    ecnt_parts = jnp.zeros((NC, NTAB, 16), jnp.float32).at[0, :N].set(
        cnt[:, None] * jnp.ones((1, 16), jnp.float32))
    ccnt = jax.ops.segment_sum(jnp.ones((N,), jnp.float32), crystal_atom_idx,
                               num_segments=NCRYS)
    ccnt_parts = jnp.zeros((NC, CTAB, 16), jnp.float32).at[0, :NCRYS].set(
        ccnt[:, None] * jnp.ones((1, 16), jnp.float32))

    x = _tc_embed(atom_fea, W_emb, b_emb)
    for i in range(4):
        Wf = W_full[i]
        xs, xn = _tc_proj(x, Wf[:FEA], Wf[FEA:2 * FEA])
        nprj = _tc_nprj(nbr_fea_p, Wf[2 * FEA:], b_full[i])
        t = _sc_gather(xs, xn, nprj, sidx_g, nidx_g)
        ac = _tc_stats(t, bn1_g[i], bn1_b[i])
        msg4 = _tc_msg(t, ac)
        msg = jnp.concatenate([msg4[q, :E] for q in range(4)], axis=1)
        s = jax.ops.segment_sum(msg, self_fea_idx, num_segments=N)
        sums = jnp.zeros((4, NTAB, FEA // 4), jnp.float32)
        for q in range(4):
            sums = sums.at[q, :N].set(s[:, q * (FEA // 4):(q + 1) * (FEA // 4)])
        x = _tc_update(sums, ecnt_parts, x, bn2_g[i], bn2_b[i])

    cs = jax.ops.segment_sum(x, crystal_atom_idx, num_segments=NCRYS)
    csums = jnp.zeros((NC, CTAB, FEA // 2), jnp.float32)
    csums = csums.at[0, :NCRYS].set(cs[:, :FEA // 2])
    csums = csums.at[1, :NCRYS].set(cs[:, FEA // 2:])
    return _tc_crys(csums, ccnt_parts)
